# baseline (speedup 1.0000x reference)
import functools

import jax
import jax.numpy as jnp
from jax import lax
from jax.experimental import pallas as pl
from jax.experimental.pallas import tpu as pltpu
from jax.experimental.pallas import tpu_sc as plsc

_L = 16


@functools.lru_cache(maxsize=None)
def _build(B, V, NN, D, n_cores, n_subcores):
    NW = n_cores * n_subcores
    G = 128
    per_w = B // NW
    NB = per_w // G
    assert per_w % G == 0 and D % _L == 0

    mesh = plsc.VectorSubcoreMesh(core_axis_name="c", subcore_axis_name="s")

    @functools.partial(
        pl.kernel,
        out_type=jax.ShapeDtypeStruct((B, D), jnp.float32),
        mesh=mesh,
        compiler_params=pltpu.CompilerParams(use_tc_tiling_on_sc=False),
        scratch_types=[
            pltpu.VMEM((per_w,), jnp.int32),
            pltpu.VMEM((2, G), jnp.int32),
            pltpu.VMEM((2, G), jnp.int32),
            pltpu.VMEM((2, G, D), jnp.float32),
            pltpu.VMEM((2, G, D), jnp.float32),
            pltpu.SemaphoreType.DMA,
            pltpu.SemaphoreType.DMA,
            pltpu.SemaphoreType.DMA,
            pltpu.SemaphoreType.DMA,
            pltpu.SemaphoreType.DMA,
            pltpu.SemaphoreType.DMA,
        ],
    )
    def k(ids_hbm, emb_hbm, comp_hbm, out_hbm,
          ids_v, idxm_v, idxc_v, rows_a, rows_b,
          sem_a0, sem_b0, sem_a1, sem_b1, sem_w0, sem_w1):
        wid = lax.axis_index("s") * n_cores + lax.axis_index("c")
        base = wid * per_w
        pltpu.sync_copy(ids_hbm.at[pl.ds(base, per_w)], ids_v)

        sems_a = (sem_a0, sem_a1)
        sems_b = (sem_b0, sem_b1)
        sems_w = (sem_w0, sem_w1)

        def prep(j, p):
            for kk in range(G // _L):
                sl = pl.ds(kk * _L, _L)
                v = ids_v[pl.ds(j * G + kk * _L, _L)]
                d = v - V
                keep = lax.shift_right_arithmetic(d, 31)
                idxm_v[p, sl] = jnp.bitwise_and(v, keep)
                idxc_v[p, sl] = jnp.bitwise_and(d, jnp.bitwise_not(keep))

        def fire(j, p):
            ca = pltpu.async_copy(emb_hbm.at[idxm_v.at[p]], rows_a.at[p], sems_a[p])
            cb = pltpu.async_copy(comp_hbm.at[idxc_v.at[p]], rows_b.at[p], sems_b[p])
            return ca, cb

        def drain(p):
            pltpu.make_async_copy(emb_hbm.at[idxm_v.at[p]], rows_a.at[p], sems_a[p]).wait()
            pltpu.make_async_copy(comp_hbm.at[idxc_v.at[p]], rows_b.at[p], sems_b[p]).wait()

        def blend(j, p):
            va = rows_a.at[p]
            vb = rows_b.at[p]
            for t in range(G // _L):
                id16 = ids_v[pl.ds(j * G + t * _L, _L)]
                for lane in range(_L):
                    lvec = jnp.full((_L,), lane, jnp.int32)
                    idsp = id16.at[lvec].get(mode="promise_in_bounds")
                    keep = lax.shift_right_arithmetic(idsp - V, 31)
                    mf = (keep + 1).astype(jnp.float32)
                    r = t * _L + lane
                    for c in range(D // _L):
                        sl = pl.ds(c * _L, _L)
                        a = va[r, sl]
                        va[r, sl] = a + mf * (vb[r, sl] - a)

        def wb_start(j, p):
            pltpu.async_copy(rows_a.at[p], out_hbm.at[pl.ds(base + j * G, G)], sems_w[p])

        def wb_wait(j, p):
            pltpu.make_async_copy(rows_a.at[p], out_hbm.at[pl.ds(base + j * G, G)], sems_w[p]).wait()

        prep(0, 0)
        fire(0, 0)

        def half(j, p, pn):
            @pl.when(j + 1 < NB)
            def _():
                prep(j + 1, pn)

            @pl.when(j >= 2)
            def _():
                wb_wait(j - 2, p)

            @pl.when(j + 1 < NB)
            def _():
                fire(j + 1, pn)

            drain(p)
            blend(j, p)
            wb_start(j, p)

        def step(jj, carry):
            half(2 * jj, 0, 1)
            half(2 * jj + 1, 1, 0)
            return carry

        assert NB % 2 == 0
        lax.fori_loop(0, NB // 2, step, 0)
        wb_wait(NB - 2, 0)
        wb_wait(NB - 1, 1)

    return k


def kernel(input_ids, emb_weight, comp_weight):
    BATCH, HIST = input_ids.shape
    V, D = emb_weight.shape
    NN = comp_weight.shape[0]
    info = plsc.get_sparse_core_info()
    ids_flat = input_ids.reshape(-1).astype(jnp.int32)
    k = _build(BATCH * HIST, V, NN, D, info.num_cores, info.num_subcores)
    out = k(ids_flat, emb_weight, comp_weight)
    return out.reshape(BATCH, HIST, D)


# speedup vs baseline: 13.3074x; 13.3074x over previous
import functools

import jax
import jax.numpy as jnp
from jax import lax
from jax.experimental import pallas as pl
from jax.experimental.pallas import tpu as pltpu
from jax.experimental.pallas import tpu_sc as plsc

_L = 16


@functools.lru_cache(maxsize=None)
def _build(B, V, NN, D, n_cores, n_subcores):
    NW = n_cores * n_subcores
    G = 128
    per_w = B // NW
    NB = per_w // G
    assert per_w % G == 0 and D % _L == 0 and NB % 2 == 0

    mesh = plsc.VectorSubcoreMesh(core_axis_name="c", subcore_axis_name="s")

    @functools.partial(
        pl.kernel,
        out_type=jax.ShapeDtypeStruct((B, D), jnp.float32),
        mesh=mesh,
        compiler_params=pltpu.CompilerParams(use_tc_tiling_on_sc=False),
        scratch_types=[
            pltpu.VMEM((per_w,), jnp.int32),
            pltpu.VMEM((4, G, D), jnp.float32),
            pltpu.SemaphoreType.DMA,
            pltpu.SemaphoreType.DMA,
            pltpu.SemaphoreType.DMA,
            pltpu.SemaphoreType.DMA,
            pltpu.SemaphoreType.DMA,
            pltpu.SemaphoreType.DMA,
            pltpu.SemaphoreType.DMA,
            pltpu.SemaphoreType.DMA,
        ],
    )
    def k(ids_hbm, emb_hbm, comp_hbm, out_hbm,
          ids_v, rows, sem_g0, sem_g1, sem_g2, sem_g3,
          sem_w0, sem_w1, sem_w2, sem_w3):
        wid = lax.axis_index("s") * n_cores + lax.axis_index("c")
        base = wid * per_w
        pltpu.sync_copy(ids_hbm.at[pl.ds(base, per_w)], ids_v)

        sems_g = (sem_g0, sem_g1, sem_g2, sem_g3)
        sems_w = (sem_w0, sem_w1, sem_w2, sem_w3)

        def fire(j, p):
            rows_p = rows.at[p]
            sem = sems_g[p]

            def grp(t, c2):
                id16 = ids_v[pl.ds(j * G + t * _L, _L)]
                for lane in range(_L):
                    rid = id16[lane]
                    d = rid - V
                    i = t * _L + lane

                    @pl.when(d < 0)
                    def _():
                        pltpu.async_copy(
                            emb_hbm.at[pl.ds(rid, 1)],
                            rows_p.at[pl.ds(i, 1)], sem)

                    @pl.when(d >= 0)
                    def _():
                        pltpu.async_copy(
                            comp_hbm.at[pl.ds(d, 1)],
                            rows_p.at[pl.ds(i, 1)], sem)
                return c2

            lax.fori_loop(0, G // _L, grp, 0)

        def drain(p):
            pltpu.make_async_copy(emb_hbm.at[pl.ds(0, G)], rows.at[p], sems_g[p]).wait()

        def wb_start(j, p):
            pltpu.async_copy(rows.at[p], out_hbm.at[pl.ds(base + j * G, G)], sems_w[p])

        def wb_wait(j, p):
            pltpu.make_async_copy(rows.at[p], out_hbm.at[pl.ds(base + j * G, G)], sems_w[p]).wait()

        fire(0, 0)

        def phase(j, p):
            pn = (p + 1) % 4

            @pl.when(j >= 3)
            def _():
                wb_wait(j - 3, pn)

            @pl.when(j + 1 < NB)
            def _():
                fire(j + 1, pn)

            drain(p)
            wb_start(j, p)

        def step(jj, carry):
            for p in range(4):
                phase(4 * jj + p, p)
            return carry

        assert NB % 4 == 0
        lax.fori_loop(0, NB // 4, step, 0)
        wb_wait(NB - 3, (NB - 3) % 4)
        wb_wait(NB - 2, (NB - 2) % 4)
        wb_wait(NB - 1, (NB - 1) % 4)

    return k


def kernel(input_ids, emb_weight, comp_weight):
    BATCH, HIST = input_ids.shape
    V, D = emb_weight.shape
    NN = comp_weight.shape[0]
    info = plsc.get_sparse_core_info()
    ids_flat = input_ids.reshape(-1).astype(jnp.int32)
    k = _build(BATCH * HIST, V, NN, D, info.num_cores, info.num_subcores)
    out = k(ids_flat, emb_weight, comp_weight)
    return out.reshape(BATCH, HIST, D)


# speedup vs baseline: 13.5734x; 1.0200x over previous
import functools

import jax
import jax.numpy as jnp
from jax import lax
from jax.experimental import pallas as pl
from jax.experimental.pallas import tpu as pltpu
from jax.experimental.pallas import tpu_sc as plsc

_L = 16


@functools.lru_cache(maxsize=None)
def _build(B, V, NN, D, n_cores, n_subcores):
    NW = n_cores * n_subcores
    G = 128
    per_w = B // NW
    NB = per_w // G
    assert per_w % G == 0 and D % _L == 0 and NB % 2 == 0

    mesh = plsc.VectorSubcoreMesh(core_axis_name="c", subcore_axis_name="s")

    @functools.partial(
        pl.kernel,
        out_type=jax.ShapeDtypeStruct((B, D), jnp.float32),
        mesh=mesh,
        compiler_params=pltpu.CompilerParams(use_tc_tiling_on_sc=True),
        scratch_types=[
            pltpu.VMEM((per_w,), jnp.int32),
            pltpu.VMEM((4, G, D), jnp.float32),
            pltpu.SemaphoreType.DMA,
            pltpu.SemaphoreType.DMA,
            pltpu.SemaphoreType.DMA,
            pltpu.SemaphoreType.DMA,
            pltpu.SemaphoreType.DMA,
            pltpu.SemaphoreType.DMA,
            pltpu.SemaphoreType.DMA,
            pltpu.SemaphoreType.DMA,
        ],
    )
    def k(ids_hbm, emb_hbm, comp_hbm, out_hbm,
          ids_v, rows, sem_g0, sem_g1, sem_g2, sem_g3,
          sem_w0, sem_w1, sem_w2, sem_w3):
        wid = lax.axis_index("s") * n_cores + lax.axis_index("c")
        base = wid * per_w
        pltpu.sync_copy(ids_hbm.at[pl.ds(base, per_w)], ids_v)

        sems_g = (sem_g0, sem_g1, sem_g2, sem_g3)
        sems_w = (sem_w0, sem_w1, sem_w2, sem_w3)

        def fire(j, p):
            rows_p = rows.at[p]
            sem = sems_g[p]

            def grp(t, c2):
                id16 = ids_v[pl.ds(j * G + t * _L, _L)]
                for lane in range(_L):
                    rid = id16[lane]
                    d = rid - V
                    i = t * _L + lane

                    @pl.when(d < 0)
                    def _():
                        pltpu.async_copy(
                            emb_hbm.at[pl.ds(rid, 1)],
                            rows_p.at[pl.ds(i, 1)], sem)

                    @pl.when(d >= 0)
                    def _():
                        pltpu.async_copy(
                            comp_hbm.at[pl.ds(d, 1)],
                            rows_p.at[pl.ds(i, 1)], sem)
                return c2

            lax.fori_loop(0, G // _L, grp, 0)

        def drain(p):
            pltpu.make_async_copy(emb_hbm.at[pl.ds(0, G)], rows.at[p], sems_g[p]).wait()

        def wb_start(j, p):
            pltpu.async_copy(rows.at[p], out_hbm.at[pl.ds(base + j * G, G)], sems_w[p])

        def wb_wait(j, p):
            pltpu.make_async_copy(rows.at[p], out_hbm.at[pl.ds(base + j * G, G)], sems_w[p]).wait()

        fire(0, 0)

        def phase(j, p):
            pn = (p + 1) % 4

            @pl.when(j >= 3)
            def _():
                wb_wait(j - 3, pn)

            @pl.when(j + 1 < NB)
            def _():
                fire(j + 1, pn)

            drain(p)
            wb_start(j, p)

        def step(jj, carry):
            for p in range(4):
                phase(4 * jj + p, p)
            return carry

        assert NB % 4 == 0
        lax.fori_loop(0, NB // 4, step, 0)
        wb_wait(NB - 3, (NB - 3) % 4)
        wb_wait(NB - 2, (NB - 2) % 4)
        wb_wait(NB - 1, (NB - 1) % 4)

    return k


def kernel(input_ids, emb_weight, comp_weight):
    BATCH, HIST = input_ids.shape
    V, D = emb_weight.shape
    NN = comp_weight.shape[0]
    info = plsc.get_sparse_core_info()
    ids_flat = input_ids.reshape(-1).astype(jnp.int32)
    k = _build(BATCH * HIST, V, NN, D, info.num_cores, info.num_subcores)
    out = k(ids_flat, emb_weight, comp_weight)
    return out.reshape(BATCH, HIST, D)
